# Initial kernel scaffold; baseline (speedup 1.0000x reference)
import jax
import jax.numpy as jnp
from jax.experimental import pallas as pl


def kernel(token_idxs, weight):
    raise NotImplementedError("write your pallas kernel here")


# speedup vs baseline: 5.0957x; 5.0957x over previous
import functools

import jax
import jax.numpy as jnp
from jax import lax
from jax.experimental import pallas as pl
from jax.experimental.pallas import tpu as pltpu
from jax.experimental.pallas import tpu_sc as plsc

BATCH = 16384
HIST = 200
D = 32
L = 128
ROWS = BATCH * HIST // L
NC = 2
NS = 16
NW = NC * NS
ROWS_PER_W = ROWS // NW
CHUNK = 8
NITER = ROWS_PER_W // CHUNK

_mesh = plsc.VectorSubcoreMesh(core_axis_name="c", subcore_axis_name="s")


@functools.partial(
    pl.kernel,
    mesh=_mesh,
    compiler_params=pltpu.CompilerParams(use_tc_tiling_on_sc=False),
    out_type=jax.ShapeDtypeStruct((ROWS, L, D), jnp.float32),
    scratch_types=[
        pltpu.VMEM((CHUNK, L), jnp.int32),
        pltpu.VMEM((CHUNK, L, D), jnp.float32),
        pltpu.SemaphoreType.DMA,
    ],
)
def _emb_lookup(idx_hbm, w_hbm, out_hbm, idx_v, rows_v, sem):
    wid = lax.axis_index("s") * NC + lax.axis_index("c")
    base = wid * ROWS_PER_W

    def body(i, carry):
        r0 = base + i * CHUNK
        pltpu.sync_copy(idx_hbm.at[pl.ds(r0, CHUNK)], idx_v)
        copies = [
            pltpu.async_copy(w_hbm.at[idx_v.at[j]], rows_v.at[j], sem)
            for j in range(CHUNK)
        ]
        for cp in copies:
            cp.wait()
        pltpu.sync_copy(rows_v, out_hbm.at[pl.ds(r0, CHUNK)])
        return carry

    lax.fori_loop(0, NITER, body, 0)


def kernel(token_idxs, weight):
    idx = token_idxs.reshape(ROWS, L)
    out = _emb_lookup(idx, weight)
    return out.reshape(BATCH, HIST, D)


# speedup vs baseline: 6.4615x; 1.2680x over previous
import functools

import jax
import jax.numpy as jnp
from jax import lax
from jax.experimental import pallas as pl
from jax.experimental.pallas import tpu as pltpu
from jax.experimental.pallas import tpu_sc as plsc

VOCAB = 1000
BATCH = 16384
HIST = 200
D = 32
L = 128
ROWS = BATCH * HIST // L
NC = 2
NS = 16
NW = NC * NS
ROWS_PER_W = ROWS // NW
CHUNK = 8
NITER = ROWS_PER_W // CHUNK

_mesh = plsc.VectorSubcoreMesh(core_axis_name="c", subcore_axis_name="s")


@functools.partial(
    pl.kernel,
    mesh=_mesh,
    compiler_params=pltpu.CompilerParams(use_tc_tiling_on_sc=False),
    out_type=jax.ShapeDtypeStruct((ROWS, L, D), jnp.float32),
    scratch_types=[
        pltpu.VMEM_SHARED((VOCAB, D), jnp.float32),
        pltpu.VMEM((CHUNK, L), jnp.int32),
        pltpu.VMEM((CHUNK, L, D), jnp.float32),
        pltpu.SemaphoreType.DMA,
    ],
)
def _emb_lookup(idx_hbm, w_hbm, out_hbm, w_sh, idx_v, rows_v, sem):
    sid = lax.axis_index("s")
    wid = sid * NC + lax.axis_index("c")
    base = wid * ROWS_PER_W

    @pl.when(sid == 0)
    def _stage_table():
        pltpu.sync_copy(w_hbm, w_sh)

    plsc.subcore_barrier()

    def body(i, carry):
        r0 = base + i * CHUNK
        pltpu.sync_copy(idx_hbm.at[pl.ds(r0, CHUNK)], idx_v)
        copies = [
            pltpu.async_copy(w_sh.at[idx_v.at[j]], rows_v.at[j], sem)
            for j in range(CHUNK)
        ]
        for cp in copies:
            cp.wait()
        pltpu.sync_copy(rows_v, out_hbm.at[pl.ds(r0, CHUNK)])
        return carry

    lax.fori_loop(0, NITER, body, 0)


def kernel(token_idxs, weight):
    idx = token_idxs.reshape(ROWS, L)
    out = _emb_lookup(idx, weight)
    return out.reshape(BATCH, HIST, D)


# speedup vs baseline: 7.0582x; 1.0924x over previous
import functools

import jax
import jax.numpy as jnp
from jax import lax
from jax.experimental import pallas as pl
from jax.experimental.pallas import tpu as pltpu
from jax.experimental.pallas import tpu_sc as plsc

VOCAB = 1000
BATCH = 16384
HIST = 200
D = 32
L = 128
ROWS = BATCH * HIST // L
NC = 2
NS = 16
NW = NC * NS
ROWS_PER_W = ROWS // NW
CHUNK = 8
NITER = ROWS_PER_W // CHUNK
NBUF = 2
NBODY = NITER // NBUF

_mesh = plsc.VectorSubcoreMesh(core_axis_name="c", subcore_axis_name="s")


@functools.partial(
    pl.kernel,
    mesh=_mesh,
    compiler_params=pltpu.CompilerParams(use_tc_tiling_on_sc=False),
    out_type=jax.ShapeDtypeStruct((ROWS, L, D), jnp.float32),
    scratch_types=[
        pltpu.VMEM_SHARED((VOCAB, D), jnp.float32),
        pltpu.VMEM((NBUF, CHUNK, L), jnp.int32),
        pltpu.VMEM((NBUF, CHUNK, L, D), jnp.float32),
        pltpu.SemaphoreType.DMA,
        [pltpu.SemaphoreType.DMA] * NBUF,
        [pltpu.SemaphoreType.DMA] * NBUF,
    ],
)
def _emb_lookup(idx_hbm, w_hbm, out_hbm, w_sh, idx_v, rows_v, gsem, ssems, isems):
    sid = lax.axis_index("s")
    wid = sid * NC + lax.axis_index("c")
    base = wid * ROWS_PER_W

    @pl.when(sid == 0)
    def _stage_table():
        pltpu.sync_copy(w_hbm, w_sh)

    plsc.subcore_barrier()

    def body(s, carry):
        for k in range(NBUF):
            i = s * NBUF + k
            r0 = base + i * CHUNK
            idx_b = idx_v.at[k]
            rows_b = rows_v.at[k]

            @pl.when(s > 0)
            def _drain_prev():
                pltpu.make_async_copy(
                    rows_b, out_hbm.at[pl.ds(r0, CHUNK)], ssems[k]
                ).wait()
                pltpu.make_async_copy(
                    idx_hbm.at[pl.ds(r0, CHUNK)], idx_b, isems[k]
                ).wait()

            @pl.when(s == 0)
            def _prime_idx():
                pltpu.sync_copy(idx_hbm.at[pl.ds(r0, CHUNK)], idx_b)

            copies = [
                pltpu.async_copy(w_sh.at[idx_b.at[j]], rows_b.at[j], gsem)
                for j in range(CHUNK)
            ]
            for cp in copies:
                cp.wait()

            @pl.when(s < NBODY - 1)
            def _prefetch_idx():
                r0n = r0 + NBUF * CHUNK
                pltpu.async_copy(idx_hbm.at[pl.ds(r0n, CHUNK)], idx_b, isems[k])

            pltpu.async_copy(rows_b, out_hbm.at[pl.ds(r0, CHUNK)], ssems[k])
        return carry

    lax.fori_loop(0, NBODY, body, 0)

    for k in range(NBUF):
        pltpu.make_async_copy(
            rows_v.at[k], out_hbm.at[pl.ds(base, CHUNK)], ssems[k]
        ).wait()


def kernel(token_idxs, weight):
    idx = token_idxs.reshape(ROWS, L)
    out = _emb_lookup(idx, weight)
    return out.reshape(BATCH, HIST, D)
